# Initial kernel scaffold; baseline (speedup 1.0000x reference)
import jax
import jax.numpy as jnp
from jax.experimental import pallas as pl


def kernel(atom_features, bond_features, connectivity, bond_transform, gru_kernel, gru_recurrent_kernel, gru_bias):
    raise NotImplementedError("write your pallas kernel here")


# speedup vs baseline: 2.7542x; 2.7542x over previous
import jax
import jax.numpy as jnp
from jax.experimental import pallas as pl
from jax.experimental.pallas import tpu as pltpu

_ATOM = 64
_BOND = 16
_B, _N, _E = 4, 512, 4096
_BP = 8


def _msg_body(atom_ref, bond_ref, src_ref, tgt_ref, bt_ref, agg_ref):
    bt16 = bt_ref[...].astype(jnp.bfloat16).astype(jnp.float32)
    w2 = jnp.sum(bt16, axis=-1)
    bond16 = bond_ref[0].astype(jnp.bfloat16).astype(jnp.float32)
    bmsg = jnp.dot(bond16, w2, preferred_element_type=jnp.float32, precision=jax.lax.Precision.HIGHEST)
    iota = jax.lax.broadcasted_iota(jnp.int32, (_E, _N), 1)
    src_oh = (iota == src_ref[0]).astype(jnp.float32)
    srows = jnp.dot(src_oh, atom_ref[0], preferred_element_type=jnp.float32, precision=jax.lax.Precision.HIGHEST)
    s = jnp.sum(srows, axis=-1, keepdims=True)
    msg = bmsg * s
    tgt_oh = (iota == tgt_ref[0]).astype(jnp.float32)
    agg = jax.lax.dot_general(tgt_oh, msg, (((0,), (0,)), ((), ())),
                              preferred_element_type=jnp.float32, precision=jax.lax.Precision.HIGHEST)
    agg_ref[0] = agg


def _gru_body(atomT_ref, aggT_ref, kg_ref, rg_ref, b_ref, out_ref,
              xz_ref, xr_ref, xh_ref):
    a = atomT_ref[...]
    g = aggT_ref[...]
    kz = kg_ref[0]
    kr = kg_ref[1]
    kh = kg_ref[2]
    xz_ref[...] = (jnp.dot(a, kz[0:_ATOM], preferred_element_type=jnp.float32)
                   + jnp.dot(g, kz[_ATOM:], preferred_element_type=jnp.float32)
                   + b_ref[0:1, :])
    xr_ref[...] = (jnp.dot(a, kr[0:_ATOM], preferred_element_type=jnp.float32)
                   + jnp.dot(g, kr[_ATOM:], preferred_element_type=jnp.float32)
                   + b_ref[1:2, :])
    xh_ref[...] = (jnp.dot(a, kh[0:_ATOM], preferred_element_type=jnp.float32)
                   + jnp.dot(g, kh[_ATOM:], preferred_element_type=jnp.float32)
                   + b_ref[2:3, :])
    rz = rg_ref[0]
    rr = rg_ref[1]
    rh = rg_ref[2]
    bz = b_ref[3:4, :]
    br = b_ref[4:5, :]
    bh = b_ref[5:6, :]

    def step(t, h):
        xz = xz_ref[pl.ds(t * _BP, _BP), :]
        xr = xr_ref[pl.ds(t * _BP, _BP), :]
        xh = xh_ref[pl.ds(t * _BP, _BP), :]
        hz = jnp.dot(h, rz, preferred_element_type=jnp.float32) + bz
        hr = jnp.dot(h, rr, preferred_element_type=jnp.float32) + br
        hh_ = jnp.dot(h, rh, preferred_element_type=jnp.float32) + bh
        z = jax.nn.sigmoid(xz + hz)
        r = jax.nn.sigmoid(xr + hr)
        hcand = jnp.tanh(xh + r * hh_)
        hn = z * h + (1.0 - z) * hcand
        out_ref[pl.ds(t * _BP, _BP), :] = hn
        return hn

    jax.lax.fori_loop(0, _N, step, jnp.zeros((_BP, _ATOM), jnp.float32))


def kernel(atom_features, bond_features, connectivity, bond_transform,
           gru_kernel, gru_recurrent_kernel, gru_bias):
    src = connectivity[:, :, 0:1]
    tgt = connectivity[:, :, 1:2]

    agg = pl.pallas_call(
        _msg_body,
        grid=(_B,),
        in_specs=[
            pl.BlockSpec((1, _N, _ATOM), lambda b: (b, 0, 0)),
            pl.BlockSpec((1, _E, _BOND), lambda b: (b, 0, 0)),
            pl.BlockSpec((1, _E, 1), lambda b: (b, 0, 0)),
            pl.BlockSpec((1, _E, 1), lambda b: (b, 0, 0)),
            pl.BlockSpec((_BOND, _ATOM, _ATOM), lambda b: (0, 0, 0)),
        ],
        out_specs=pl.BlockSpec((1, _N, _ATOM), lambda b: (b, 0, 0)),
        out_shape=jax.ShapeDtypeStruct((_B, _N, _ATOM), jnp.float32),
    )(atom_features, bond_features, src, tgt, bond_transform)

    atomT = jnp.zeros((_N, _BP, _ATOM), jnp.float32)
    atomT = atomT.at[:, :_B].set(jnp.swapaxes(atom_features, 0, 1))
    aggT = jnp.zeros((_N, _BP, _ATOM), jnp.float32)
    aggT = aggT.at[:, :_B].set(jnp.swapaxes(agg, 0, 1))
    atomT2 = atomT.reshape(_N * _BP, _ATOM)
    aggT2 = aggT.reshape(_N * _BP, _ATOM)

    kg = jnp.swapaxes(gru_kernel.reshape(2 * _ATOM, 3, _ATOM), 0, 1)
    rg = jnp.swapaxes(gru_recurrent_kernel.reshape(_ATOM, 3, _ATOM), 0, 1)
    bg = gru_bias.reshape(6, _ATOM)

    out2 = pl.pallas_call(
        _gru_body,
        out_shape=jax.ShapeDtypeStruct((_N * _BP, _ATOM), jnp.float32),
        scratch_shapes=[
            pltpu.VMEM((_N * _BP, _ATOM), jnp.float32),
            pltpu.VMEM((_N * _BP, _ATOM), jnp.float32),
            pltpu.VMEM((_N * _BP, _ATOM), jnp.float32),
        ],
    )(atomT2, aggT2, kg, rg, bg)

    out = out2.reshape(_N, _BP, _ATOM)[:, :_B]
    return jnp.swapaxes(out, 0, 1)
